# Initial kernel scaffold; baseline (speedup 1.0000x reference)
import jax
import jax.numpy as jnp
from jax.experimental import pallas as pl


def kernel(x, edge_index, W0l, b0l, W0r, gamma0, beta0, W1l, b1l, W1r):
    raise NotImplementedError("write your pallas kernel here")


# speedup vs baseline: 2.9513x; 2.9513x over previous
import functools

import jax
import jax.numpy as jnp
from jax import lax
from jax.experimental import pallas as pl
from jax.experimental.pallas import tpu as pltpu
from jax.experimental.pallas import tpu_sc as plsc

N = 10000
E = 320000
D = 128

NC = 2
NS = 16
NW = NC * NS
EPT = 10240
E_PAD = NW * EPT
EB = 128
NB = EPT // EB
N_PAD = 10240
R = 1000


def _sc_agg_body(tab_hbm, src_hbm, dst_hbm, zeros_hbm, iota_hbm,
                 parts_hbm,
                 src_i, dst_i, idx_v, rows_v, agg_sh, sem):
    cid = lax.axis_index("c")
    sid = lax.axis_index("s")
    tid = cid * NS + sid

    rows_per_sub = N_PAD // NS
    base = sid * rows_per_sub
    nchunk = rows_per_sub // EB

    pltpu.sync_copy(zeros_hbm.at[pl.ds(0, EB)], rows_v)
    for j in range(nchunk):
        pltpu.sync_copy(iota_hbm.at[pl.ds(base + j * EB, EB)], idx_v)
        pltpu.sync_copy(rows_v, agg_sh.at[idx_v])

    plsc.subcore_barrier()

    ebase = tid * EPT

    def _block(m, c):
        pltpu.sync_copy(src_hbm.at[pl.ds(ebase + m * EB, EB)], src_i)
        pltpu.sync_copy(dst_hbm.at[pl.ds(ebase + m * EB, EB)], dst_i)
        pltpu.async_copy(tab_hbm.at[src_i], rows_v, sem).wait()
        pltpu.sync_copy(rows_v, agg_sh.at[dst_i], add=True)
        return c
    lax.fori_loop(0, NB, _block, 0)

    plsc.subcore_barrier()

    for j in range(nchunk):
        pltpu.sync_copy(iota_hbm.at[pl.ds(base + j * EB, EB)], idx_v)
        ochunk = pl.ds(cid * N_PAD + base + j * EB, EB)
        pltpu.async_copy(agg_sh.at[idx_v], rows_v, sem).wait()
        pltpu.sync_copy(rows_v, parts_hbm.at[ochunk])


def _sc_aggregate(tab, src_p, dst_p, zeros, iota):
    mesh = plsc.VectorSubcoreMesh(core_axis_name="c", subcore_axis_name="s")
    f = pl.kernel(
        _sc_agg_body,
        out_type=jax.ShapeDtypeStruct((NC * N_PAD, D), jnp.float32),
        mesh=mesh,
        scratch_types=[
            pltpu.VMEM((EB,), jnp.int32),
            pltpu.VMEM((EB,), jnp.int32),
            pltpu.VMEM((EB,), jnp.int32),
            pltpu.VMEM((EB, D), jnp.float32),
            pltpu.VMEM_SHARED((N_PAD, D), jnp.float32),
            pltpu.SemaphoreType.DMA,
        ],
    )
    parts = f(tab, src_p, dst_p, zeros, iota)
    return parts.reshape(NC, N_PAD, D)


def _sc_degree_body(dst_hbm, ones_hbm, zeros_hbm, iota_hbm, degp_hbm,
                    dst_i, idx_v, ones_v, deg_sh, sem):
    cid = lax.axis_index("c")
    sid = lax.axis_index("s")
    tid = cid * NS + sid

    rows_per_sub = N_PAD // NS
    base = sid * rows_per_sub
    nchunk = rows_per_sub // EB

    pltpu.sync_copy(zeros_hbm.at[pl.ds(0, EB)], ones_v)
    for j in range(nchunk):
        pltpu.sync_copy(iota_hbm.at[pl.ds(base + j * EB, EB)], idx_v)
        pltpu.sync_copy(ones_v, deg_sh.at[idx_v])
    pltpu.sync_copy(ones_hbm, ones_v)

    plsc.subcore_barrier()

    ebase = tid * EPT

    def _block(m, c):
        pltpu.sync_copy(dst_hbm.at[pl.ds(ebase + m * EB, EB)], dst_i)
        pltpu.sync_copy(ones_v, deg_sh.at[dst_i], add=True)
        return c
    lax.fori_loop(0, NB, _block, 0)

    plsc.subcore_barrier()

    for j in range(nchunk):
        pltpu.sync_copy(iota_hbm.at[pl.ds(base + j * EB, EB)], idx_v)
        ochunk = pl.ds(cid * N_PAD + base + j * EB, EB)
        pltpu.async_copy(deg_sh.at[idx_v], ones_v, sem).wait()
        pltpu.sync_copy(ones_v, degp_hbm.at[ochunk])


def _sc_degree(dst_p, ones, zeros, iota):
    mesh = plsc.VectorSubcoreMesh(core_axis_name="c", subcore_axis_name="s")
    f = pl.kernel(
        _sc_degree_body,
        out_type=jax.ShapeDtypeStruct((NC * N_PAD, D), jnp.float32),
        mesh=mesh,
        scratch_types=[
            pltpu.VMEM((EB,), jnp.int32),
            pltpu.VMEM((EB,), jnp.int32),
            pltpu.VMEM((EB, D), jnp.float32),
            pltpu.VMEM_SHARED((N_PAD, D), jnp.float32),
            pltpu.SemaphoreType.DMA,
        ],
    )
    return f(dst_p, ones, zeros, iota).reshape(NC, N_PAD, D)


def _dense_body(with_stats, parts_ref, degp_ref, x_ref, wl_ref, bl_ref,
                wr_ref, *refs):
    agg = parts_ref[0] + parts_ref[1]
    deg = jnp.maximum(degp_ref[0, :, 0:1] + degp_ref[1, :, 0:1], 1.0)
    mean = agg / deg
    h = (lax.dot_general(mean, wl_ref[...], (((1,), (1,)), ((), ())),
                         preferred_element_type=jnp.float32)
         + bl_ref[...]
         + lax.dot_general(x_ref[...], wr_ref[...], (((1,), (1,)), ((), ())),
                           preferred_element_type=jnp.float32))
    if with_stats:
        out_ref, sum_ref, sumsq_ref = refs
        out_ref[...] = h

        @pl.when(pl.program_id(0) == 0)
        def _():
            sum_ref[...] = jnp.zeros_like(sum_ref)
            sumsq_ref[...] = jnp.zeros_like(sumsq_ref)
        sum_ref[...] += h.sum(axis=0, keepdims=True)
        sumsq_ref[...] += (h * h).sum(axis=0, keepdims=True)
    else:
        (out_ref,) = refs
        out_ref[...] = h


def _dense_layer(parts, degp, x, wl, bl, wr, with_stats):
    nblk = N // R
    out_shapes = [jax.ShapeDtypeStruct((N, D), jnp.float32)]
    out_specs = [pl.BlockSpec((R, D), lambda i: (i, 0))]
    if with_stats:
        out_shapes += [jax.ShapeDtypeStruct((1, D), jnp.float32)] * 2
        out_specs += [pl.BlockSpec((1, D), lambda i: (0, 0))] * 2
    return pl.pallas_call(
        functools.partial(_dense_body, with_stats),
        grid=(nblk,),
        in_specs=[
            pl.BlockSpec((NC, R, D), lambda i: (0, i, 0)),
            pl.BlockSpec((NC, R, D), lambda i: (0, i, 0)),
            pl.BlockSpec((R, D), lambda i: (i, 0)),
            pl.BlockSpec((D, D), lambda i: (0, 0)),
            pl.BlockSpec((1, D), lambda i: (0, 0)),
            pl.BlockSpec((D, D), lambda i: (0, 0)),
        ],
        out_specs=out_specs,
        out_shape=out_shapes,
    )(parts, degp, x, wl, bl, wr)


def _bn_relu_body(h_ref, sum_ref, sumsq_ref, gamma_ref, beta_ref, out_ref):
    mu = sum_ref[...] / N
    var = sumsq_ref[...] / N - mu * mu
    inv = lax.rsqrt(var + 1e-5)
    out_ref[...] = jnp.maximum(
        gamma_ref[...] * (h_ref[...] - mu) * inv + beta_ref[...], 0.0)


def _bn_relu(h, s, s2, gamma, beta):
    return pl.pallas_call(
        _bn_relu_body,
        grid=(N // R,),
        in_specs=[
            pl.BlockSpec((R, D), lambda i: (i, 0)),
            pl.BlockSpec((1, D), lambda i: (0, 0)),
            pl.BlockSpec((1, D), lambda i: (0, 0)),
            pl.BlockSpec((1, D), lambda i: (0, 0)),
            pl.BlockSpec((1, D), lambda i: (0, 0)),
        ],
        out_specs=pl.BlockSpec((R, D), lambda i: (i, 0)),
        out_shape=jax.ShapeDtypeStruct((N, D), jnp.float32),
    )(h, s, s2, gamma, beta)


def kernel(x, edge_index, W0l, b0l, W0r, gamma0, beta0, W1l, b1l, W1r):
    src = edge_index[0]
    dst = edge_index[1]
    pad = E_PAD - E
    src_p = jnp.concatenate([src, jnp.zeros((pad,), jnp.int32)])
    dst_p = jnp.concatenate(
        [dst, N + (jnp.arange(pad, dtype=jnp.int32) % (N_PAD - N))])
    zeros = jnp.zeros((N_PAD, D), jnp.float32)
    ones = jnp.ones((EB, D), jnp.float32)
    iota = jnp.arange(N_PAD, dtype=jnp.int32)

    bl0 = b0l.reshape(1, D)
    bl1 = b1l.reshape(1, D)
    g0 = gamma0.reshape(1, D)
    be0 = beta0.reshape(1, D)

    degp = _sc_degree(dst_p, ones, zeros, iota)
    parts0 = _sc_aggregate(x, src_p, dst_p, zeros, iota)
    h, s, s2 = _dense_layer(parts0, degp, x, W0l, bl0, W0r, True)
    h_act = _bn_relu(h, s, s2, g0, be0)
    parts1 = _sc_aggregate(h_act, src_p, dst_p, zeros, iota)
    out = _dense_layer(parts1, degp, h_act, W1l, bl1, W1r, False)
    return out[0]
